# baseline (speedup 1.0000x reference)
import functools

import jax
import jax.numpy as jnp
from jax import lax
from jax.experimental import pallas as pl
from jax.experimental.pallas import tpu as pltpu
from jax.experimental.pallas import tpu_sc as plsc

BH = 16
BW = 16
HH = BH // 2


def _make_sc_gather(nB, C, rows_total):
    info = plsc.get_sparse_core_info()
    NC, NS = info.num_cores, info.num_subcores
    NW = NC * NS
    nblk = nB // NW
    half_rows = HH * C
    half_elems = HH * BW * C
    out_elems = BH * BW * C
    row_elems = BW * C
    span = (BW - 1) * C + 1

    mesh = plsc.VectorSubcoreMesh(core_axis_name="c", subcore_axis_name="s")

    @functools.partial(
        pl.kernel,
        mesh=mesh,
        compiler_params=pltpu.CompilerParams(
            needs_layout_passes=False, use_tc_tiling_on_sc=False
        ),
        out_type=jax.ShapeDtypeStruct((nB, out_elems), jnp.float32),
        scratch_types=[
            pltpu.VMEM((2, half_rows), jnp.int32),
            pltpu.VMEM((2, half_rows), jnp.int32),
            pltpu.VMEM((half_rows, 16), jnp.float32),
            pltpu.VMEM((half_rows, 16), jnp.float32),
            pltpu.VMEM((half_elems,), jnp.float32),
            pltpu.VMEM((half_elems,), jnp.float32),
            pltpu.SemaphoreType.DMA,
            pltpu.SemaphoreType.DMA,
            pltpu.SemaphoreType.DMA,
            pltpu.SemaphoreType.DMA,
            pltpu.SemaphoreType.DMA,
            pltpu.SemaphoreType.DMA,
        ],
    )
    def k(table_hbm, idx_hbm, out_hbm, idxA, idxB, bufA, bufB, outA, outB,
          semA, semB, wsemA, wsemB, isemA, isemB):
        wid = lax.axis_index("s") * NC + lax.axis_index("c")
        i0 = wid * nblk
        last = i0 + nblk - 1
        iotaC = lax.iota(jnp.int32, 16) * C

        def fire(idx_ref, h, buf_ref, sem):
            pltpu.async_copy(table_hbm.at[idx_ref.at[h]], buf_ref, sem)

        def drain_gather(idx_ref, buf_ref, sem):
            pltpu.make_async_copy(
                table_hbm.at[idx_ref.at[0]], buf_ref, sem
            ).wait()

        def transpose(buf_ref, out_ref):
            def a_body(a, carry):
                rbase = a * C
                sbase = a * row_elems

                @plsc.parallel_loop(0, C, unroll=16)
                def tr(c):
                    v = buf_ref[rbase + c, :]
                    plsc.store_scatter(out_ref, [iotaC + (sbase + c)], v)

                return carry

            lax.fori_loop(0, HH, a_body, 0)

        def issue_write(out_ref, i, h, wsem):
            pltpu.async_copy(
                out_ref, out_hbm.at[i, pl.ds(h * half_elems, half_elems)], wsem
            )

        def drain_write(out_ref, wsem):
            pltpu.make_async_copy(
                out_ref, out_hbm.at[0, pl.ds(0, half_elems)], wsem
            ).wait()

        def idx_fetch(i, idx_ref, isem):
            pltpu.async_copy(idx_hbm.at[i], idx_ref, isem)

        def idx_wait(idx_ref, isem):
            pltpu.make_async_copy(idx_hbm.at[0], idx_ref, isem).wait()

        def halfstep(idx_ref, buf_ref, sem, out_ref, wsem, i, h, t):
            drain_gather(idx_ref, buf_ref, sem)

            @pl.when(t > 0)
            def _():
                drain_write(out_ref, wsem)

            transpose(buf_ref, out_ref)
            issue_write(out_ref, i, h, wsem)

        pltpu.sync_copy(idx_hbm.at[i0], idxA)
        fire(idxA, 0, bufA, semA)
        idx_fetch(i0 + 1, idxB, isemB)

        def body(t, carry):
            b0 = i0 + 2 * t
            b1 = b0 + 1
            b2 = jnp.minimum(b1 + 1, last)
            b3 = jnp.minimum(b2 + 1, last)

            fire(idxA, 1, bufB, semB)
            halfstep(idxA, bufA, semA, outA, wsemA, b0, 0, t)
            idx_wait(idxB, isemB)
            fire(idxB, 0, bufA, semA)
            halfstep(idxA, bufB, semB, outB, wsemB, b0, 1, t)
            idx_fetch(b2, idxA, isemA)
            fire(idxB, 1, bufB, semB)
            halfstep(idxB, bufA, semA, outA, wsemA, b1, 0, t + 1)
            idx_wait(idxA, isemA)
            fire(idxA, 0, bufA, semA)
            halfstep(idxB, bufB, semB, outB, wsemB, b1, 1, t + 1)
            idx_fetch(b3, idxB, isemB)
            return carry

        lax.fori_loop(0, nblk // 2, body, 0)

        drain_gather(idxA, bufA, semA)
        idx_wait(idxB, isemB)
        drain_write(outA, wsemA)
        drain_write(outB, wsemB)

    return k


def kernel(x, indices, block_size, block_stride, block_offset):
    N, C, H, W = x.shape
    nB = indices.shape[0]
    wc = W // BW
    rows_total = N * C * H * wc

    n = indices[:, 0]
    ys = indices[:, 1] * block_stride[0] + block_offset[0]
    ws = (indices[:, 2] * block_stride[1] + block_offset[1]) // BW
    base = n * (C * H * wc) + ys * wc + ws
    a_off = jnp.arange(BH, dtype=jnp.int32) * wc
    c_off = jnp.arange(C, dtype=jnp.int32) * (H * wc)
    idx_all = (base[:, None, None] + a_off[None, :, None]
               + c_off[None, None, :]).astype(jnp.int32)

    table = x.reshape(rows_total, BW)
    idx_all = idx_all.reshape(nB, 2, HH * C)
    out = _make_sc_gather(nB, C, rows_total)(table, idx_all)
    return out.reshape(nB, BH, BW, C)


# speedup vs baseline: 1.6270x; 1.6270x over previous
import functools

import jax
import jax.numpy as jnp
from jax import lax
from jax.experimental import pallas as pl
from jax.experimental.pallas import tpu as pltpu
from jax.experimental.pallas import tpu_sc as plsc

BH = 16
BW = 16
HH = BH // 2


def _make_sc_gather(nB, C, rows_total):
    info = plsc.get_sparse_core_info()
    NC, NS = info.num_cores, info.num_subcores
    NW = NC * NS
    nblk = nB // NW
    half_rows = HH * C
    half_elems = HH * BW * C
    out_elems = BH * BW * C
    row_elems = BW * C
    span = (BW - 1) * C + 1

    mesh = plsc.VectorSubcoreMesh(core_axis_name="c", subcore_axis_name="s")

    @functools.partial(
        pl.kernel,
        mesh=mesh,
        compiler_params=pltpu.CompilerParams(
            needs_layout_passes=False, use_tc_tiling_on_sc=False
        ),
        out_type=jax.ShapeDtypeStruct((nB, out_elems), jnp.float32),
        scratch_types=[
            pltpu.VMEM((2, half_rows), jnp.int32),
            pltpu.VMEM((2, half_rows), jnp.int32),
            pltpu.VMEM((half_rows, 16), jnp.float32),
            pltpu.VMEM((half_rows, 16), jnp.float32),
            pltpu.VMEM((half_elems,), jnp.float32),
            pltpu.VMEM((half_elems,), jnp.float32),
            pltpu.SemaphoreType.DMA,
            pltpu.SemaphoreType.DMA,
            pltpu.SemaphoreType.DMA,
            pltpu.SemaphoreType.DMA,
            pltpu.SemaphoreType.DMA,
            pltpu.SemaphoreType.DMA,
        ],
    )
    def k(table_hbm, idx_hbm, out_hbm, idxA, idxB, bufA, bufB, outA, outB,
          semA, semB, wsemA, wsemB, isemA, isemB):
        wid = lax.axis_index("s") * NC + lax.axis_index("c")
        i0 = wid * nblk
        last = i0 + nblk - 1
        iotaC = lax.iota(jnp.int32, 16) * C

        def fire(idx_ref, h, buf_ref, sem):
            pltpu.async_copy(table_hbm.at[idx_ref.at[h]], buf_ref, sem)

        def drain_gather(idx_ref, buf_ref, sem):
            pltpu.make_async_copy(
                table_hbm.at[idx_ref.at[0]], buf_ref, sem
            ).wait()

        def transpose(buf_ref, out_ref):
            def a_body(a, carry):
                rbase = a * C
                sbase = a * row_elems

                @plsc.parallel_loop(0, C, unroll=16)
                def tr(c):
                    v = buf_ref[rbase + c, :]
                    plsc.store_scatter(out_ref, [iotaC + (sbase + c)], v)

                return carry

            lax.fori_loop(0, HH, a_body, 0)

        def issue_write(out_ref, i, h, wsem):
            pltpu.async_copy(
                out_ref, out_hbm.at[i, pl.ds(h * half_elems, half_elems)], wsem
            )

        def drain_write(out_ref, wsem):
            pltpu.make_async_copy(
                out_ref, out_hbm.at[0, pl.ds(0, half_elems)], wsem
            ).wait()

        def idx_fetch(i, idx_ref, isem):
            pltpu.async_copy(idx_hbm.at[i], idx_ref, isem)

        def idx_wait(idx_ref, isem):
            pltpu.make_async_copy(idx_hbm.at[0], idx_ref, isem).wait()

        def halfstep(idx_ref, buf_ref, sem, out_ref, wsem, i, h, t):
            drain_gather(idx_ref, buf_ref, sem)

            @pl.when(t > 0)
            def _():
                drain_write(out_ref, wsem)

            transpose(buf_ref, out_ref)
            issue_write(out_ref, i, h, wsem)

        pltpu.sync_copy(idx_hbm.at[i0], idxA)
        fire(idxA, 0, bufA, semA)
        idx_fetch(i0 + 1, idxB, isemB)

        def body(t, carry):
            b0 = i0 + 2 * t
            b1 = b0 + 1
            b2 = jnp.minimum(b1 + 1, last)
            b3 = jnp.minimum(b2 + 1, last)

            fire(idxA, 1, bufB, semB)
            halfstep(idxA, bufA, semA, outA, wsemA, b0, 0, t)
            idx_wait(idxB, isemB)
            fire(idxB, 0, bufA, semA)
            halfstep(idxA, bufB, semB, outB, wsemB, b0, 1, t)
            idx_fetch(b2, idxA, isemA)
            fire(idxB, 1, bufB, semB)
            halfstep(idxB, bufA, semA, outA, wsemA, b1, 0, t + 1)
            idx_wait(idxA, isemA)
            fire(idxA, 0, bufA, semA)
            halfstep(idxB, bufB, semB, outB, wsemB, b1, 1, t + 1)
            idx_fetch(b3, idxB, isemB)
            return carry

        lax.fori_loop(0, nblk // 2, body, 0)

        drain_gather(idxA, bufA, semA)
        idx_wait(idxB, isemB)
        drain_write(outA, wsemA)
        drain_write(outB, wsemB)

    return k


def kernel(x, indices, block_size, block_stride, block_offset):
    N, C, H, W = x.shape
    nB = indices.shape[0]
    wc = W // BW
    rows_total = N * C * H * wc

    n = indices[:, 0]
    ys = indices[:, 1] * block_stride[0] + block_offset[0]
    xs = indices[:, 2] * block_stride[1] + block_offset[1]

    h = ys[:, None] + jnp.arange(BH, dtype=jnp.int32)[None, :]
    tr = h >> 3
    hi = h & 7
    tc = (xs >> 7)[:, None]
    ck = ((xs & 127) >> 4)[:, None]
    prow16 = tr * ((W // 128) * 64) + hi * 8 + tc * 64 + ck
    plane = (n[:, None, None] * C
             + jnp.arange(C, dtype=jnp.int32)[None, None, :])
    idx_all = (plane * (H * wc)
               + prow16[:, :, None]).astype(jnp.int32)

    table = (x.reshape(N * C, H // 8, 8, W // 128, 128)
             .transpose(0, 1, 3, 2, 4)
             .reshape(rows_total, BW))
    idx_all = idx_all.reshape(nB, 2, HH * C)
    out = _make_sc_gather(nB, C, rows_total)(table, idx_all)
    return out.reshape(nB, BH, BW, C)


# speedup vs baseline: 4.7951x; 2.9472x over previous
import functools

import jax
import jax.numpy as jnp
from jax import lax
from jax.experimental import pallas as pl
from jax.experimental.pallas import tpu as pltpu
from jax.experimental.pallas import tpu_sc as plsc

BH = 16
BW = 16
HH = BH // 2


def _make_sc_gather(nB, C, rows_total):
    info = plsc.get_sparse_core_info()
    NC, NS = info.num_cores, info.num_subcores
    NW = NC * NS
    nblk = nB // NW
    half_rows = HH * C
    half_elems = HH * BW * C
    out_elems = BH * BW * C
    row_elems = BW * C
    span = (BW - 1) * C + 1

    mesh = plsc.VectorSubcoreMesh(core_axis_name="c", subcore_axis_name="s")

    @functools.partial(
        pl.kernel,
        mesh=mesh,
        compiler_params=pltpu.CompilerParams(
            needs_layout_passes=False, use_tc_tiling_on_sc=False
        ),
        out_type=jax.ShapeDtypeStruct((nB, 2, HH * BW, C), jnp.float32),
        scratch_types=[
            pltpu.VMEM((2, half_rows), jnp.int32),
            pltpu.VMEM((2, half_rows), jnp.int32),
            pltpu.VMEM((half_rows, 16), jnp.float32),
            pltpu.VMEM((half_rows, 16), jnp.float32),
            pltpu.VMEM((HH * BW, 129), jnp.float32),
            pltpu.VMEM((HH * BW, 129), jnp.float32),
            pltpu.SemaphoreType.DMA,
            pltpu.SemaphoreType.DMA,
            pltpu.SemaphoreType.DMA,
            pltpu.SemaphoreType.DMA,
            pltpu.SemaphoreType.DMA,
            pltpu.SemaphoreType.DMA,
        ],
    )
    def k(table_hbm, idx_hbm, out_hbm, idxA, idxB, bufA, bufB, outA, outB,
          semA, semB, wsemA, wsemB, isemA, isemB):
        wid = lax.axis_index("s") * NC + lax.axis_index("c")
        i0 = wid * nblk
        last = i0 + nblk - 1
        iota16 = lax.iota(jnp.int32, 16)

        def fire(idx_ref, h, buf_ref, sem):
            pltpu.async_copy(table_hbm.at[idx_ref.at[h]], buf_ref, sem)

        def drain_gather(idx_ref, buf_ref, sem):
            pltpu.make_async_copy(
                table_hbm.at[idx_ref.at[0]], buf_ref, sem
            ).wait()

        def transpose(buf_ref, out_ref):
            def a_body(a, carry):
                rbase = a * C
                rowv = iota16 + a * BW

                @plsc.parallel_loop(0, C, unroll=16)
                def tr(c):
                    v = buf_ref[rbase + c, :]
                    colv = jnp.broadcast_to(c, (16,))
                    plsc.store_scatter(out_ref, [rowv, colv], v)

                return carry

            lax.fori_loop(0, HH, a_body, 0)

        def issue_write(out_ref, i, h, wsem):
            pltpu.async_copy(
                out_ref.at[:, pl.ds(0, C)], out_hbm.at[i, h], wsem
            )

        def drain_write(out_ref, wsem):
            pltpu.make_async_copy(
                out_ref.at[:, pl.ds(0, C)], out_hbm.at[0, 0], wsem
            ).wait()

        def idx_fetch(i, idx_ref, isem):
            pltpu.async_copy(idx_hbm.at[i], idx_ref, isem)

        def idx_wait(idx_ref, isem):
            pltpu.make_async_copy(idx_hbm.at[0], idx_ref, isem).wait()

        def halfstep(idx_ref, buf_ref, sem, out_ref, wsem, i, h, t):
            drain_gather(idx_ref, buf_ref, sem)

            @pl.when(t > 0)
            def _():
                drain_write(out_ref, wsem)

            transpose(buf_ref, out_ref)
            issue_write(out_ref, i, h, wsem)

        pltpu.sync_copy(idx_hbm.at[i0], idxA)
        fire(idxA, 0, bufA, semA)
        idx_fetch(i0 + 1, idxB, isemB)

        def body(t, carry):
            b0 = i0 + 2 * t
            b1 = b0 + 1
            b2 = jnp.minimum(b1 + 1, last)
            b3 = jnp.minimum(b2 + 1, last)

            fire(idxA, 1, bufB, semB)
            halfstep(idxA, bufA, semA, outA, wsemA, b0, 0, t)
            idx_wait(idxB, isemB)
            fire(idxB, 0, bufA, semA)
            halfstep(idxA, bufB, semB, outB, wsemB, b0, 1, t)
            idx_fetch(b2, idxA, isemA)
            fire(idxB, 1, bufB, semB)
            halfstep(idxB, bufA, semA, outA, wsemA, b1, 0, t + 1)
            idx_wait(idxA, isemA)
            fire(idxA, 0, bufA, semA)
            halfstep(idxB, bufB, semB, outB, wsemB, b1, 1, t + 1)
            idx_fetch(b3, idxB, isemB)
            return carry

        lax.fori_loop(0, nblk // 2, body, 0)

        drain_gather(idxA, bufA, semA)
        idx_wait(idxB, isemB)
        drain_write(outA, wsemA)
        drain_write(outB, wsemB)

    return k


def kernel(x, indices, block_size, block_stride, block_offset):
    N, C, H, W = x.shape
    nB = indices.shape[0]
    wc = W // BW
    rows_total = N * C * H * wc

    n = indices[:, 0]
    ys = indices[:, 1] * block_stride[0] + block_offset[0]
    xs = indices[:, 2] * block_stride[1] + block_offset[1]

    h = ys[:, None] + jnp.arange(BH, dtype=jnp.int32)[None, :]
    tr = h >> 3
    hi = h & 7
    tc = (xs >> 7)[:, None]
    ck = ((xs & 127) >> 4)[:, None]
    prow16 = tr * ((W // 128) * 64) + hi * 8 + tc * 64 + ck
    plane = (n[:, None, None] * C
             + jnp.arange(C, dtype=jnp.int32)[None, None, :])
    idx_all = (plane * (H * wc)
               + prow16[:, :, None]).astype(jnp.int32)

    table = (x.reshape(N * C, H // 8, 8, W // 128, 128)
             .transpose(0, 1, 3, 2, 4)
             .reshape(rows_total, BW))
    idx_all = idx_all.reshape(nB, 2, HH * C)
    out = _make_sc_gather(nB, C, rows_total)(table, idx_all)
    return out.reshape(nB, BH, BW, C)


# speedup vs baseline: 5.1365x; 1.0712x over previous
import functools

import jax
import jax.numpy as jnp
from jax import lax
from jax.experimental import pallas as pl
from jax.experimental.pallas import tpu as pltpu
from jax.experimental.pallas import tpu_sc as plsc

BH = 16
BW = 16
HH = BH // 2


def _make_sc_gather(nB, C, rows_total):
    info = plsc.get_sparse_core_info()
    NC, NS = info.num_cores, info.num_subcores
    NW = NC * NS
    nblk = nB // NW
    half_rows = HH * C
    half_elems = HH * BW * C
    out_elems = BH * BW * C
    row_elems = BW * C
    span = (BW - 1) * C + 1

    mesh = plsc.VectorSubcoreMesh(core_axis_name="c", subcore_axis_name="s")

    @functools.partial(
        pl.kernel,
        mesh=mesh,
        compiler_params=pltpu.CompilerParams(
            needs_layout_passes=False, use_tc_tiling_on_sc=False
        ),
        out_type=jax.ShapeDtypeStruct((nB, 2, HH * BW, C), jnp.float32),
        scratch_types=[
            pltpu.VMEM((2, half_rows), jnp.int32),
            pltpu.VMEM((2, half_rows), jnp.int32),
            pltpu.VMEM((half_rows, 16), jnp.float32),
            pltpu.VMEM((half_rows, 16), jnp.float32),
            pltpu.VMEM((HH * BW, 129), jnp.float32),
            pltpu.VMEM((HH * BW, 129), jnp.float32),
            pltpu.SemaphoreType.DMA,
            pltpu.SemaphoreType.DMA,
            pltpu.SemaphoreType.DMA,
            pltpu.SemaphoreType.DMA,
            pltpu.SemaphoreType.DMA,
            pltpu.SemaphoreType.DMA,
        ],
    )
    def k(table_hbm, idx_hbm, out_hbm, idxA, idxB, bufA, bufB, outA, outB,
          semA, semB, wsemA, wsemB, isemA, isemB):
        wid = lax.axis_index("s") * NC + lax.axis_index("c")
        i0 = wid * nblk
        last = i0 + nblk - 1
        iota16 = lax.iota(jnp.int32, 16)

        def fire(idx_ref, h, buf_ref, sem):
            pltpu.async_copy(table_hbm.at[idx_ref.at[h]], buf_ref, sem)

        def drain_gather(idx_ref, buf_ref, sem):
            pltpu.make_async_copy(
                table_hbm.at[idx_ref.at[0]], buf_ref, sem
            ).wait()

        iota129 = iota16 * 129
        zero16 = jnp.zeros((16,), jnp.int32)

        def transpose(buf_ref, out_ref):
            def a_body(a, carry):
                rbase = a * C
                idx0 = iota129 + a * (BW * 129)

                @plsc.parallel_loop(0, C, unroll=16, carry=idx0)
                def tr(c, idxv):
                    v = buf_ref[rbase + c, :]
                    plsc.store_scatter(out_ref, [zero16, idxv], v)
                    return idxv + 1

                return carry

            lax.fori_loop(0, HH, a_body, 0)

        def issue_write(out_ref, i, h, wsem):
            pltpu.async_copy(
                out_ref.at[:, pl.ds(0, C)], out_hbm.at[i, h], wsem
            )

        def drain_write(out_ref, wsem):
            pltpu.make_async_copy(
                out_ref.at[:, pl.ds(0, C)], out_hbm.at[0, 0], wsem
            ).wait()

        def idx_fetch(i, idx_ref, isem):
            pltpu.async_copy(idx_hbm.at[i], idx_ref, isem)

        def idx_wait(idx_ref, isem):
            pltpu.make_async_copy(idx_hbm.at[0], idx_ref, isem).wait()

        def halfstep(idx_ref, buf_ref, sem, out_ref, wsem, i, h, t):
            drain_gather(idx_ref, buf_ref, sem)

            @pl.when(t > 0)
            def _():
                drain_write(out_ref, wsem)

            transpose(buf_ref, out_ref)
            issue_write(out_ref, i, h, wsem)

        pltpu.sync_copy(idx_hbm.at[i0], idxA)
        fire(idxA, 0, bufA, semA)
        idx_fetch(i0 + 1, idxB, isemB)

        def body(t, carry):
            b0 = i0 + 2 * t
            b1 = b0 + 1
            b2 = jnp.minimum(b1 + 1, last)
            b3 = jnp.minimum(b2 + 1, last)

            fire(idxA, 1, bufB, semB)
            halfstep(idxA, bufA, semA, outA, wsemA, b0, 0, t)
            idx_wait(idxB, isemB)
            fire(idxB, 0, bufA, semA)
            halfstep(idxA, bufB, semB, outB, wsemB, b0, 1, t)
            idx_fetch(b2, idxA, isemA)
            fire(idxB, 1, bufB, semB)
            halfstep(idxB, bufA, semA, outA, wsemA, b1, 0, t + 1)
            idx_wait(idxA, isemA)
            fire(idxA, 0, bufA, semA)
            halfstep(idxB, bufB, semB, outB, wsemB, b1, 1, t + 1)
            idx_fetch(b3, idxB, isemB)
            return carry

        lax.fori_loop(0, nblk // 2, body, 0)

        drain_gather(idxA, bufA, semA)
        idx_wait(idxB, isemB)
        drain_write(outA, wsemA)
        drain_write(outB, wsemB)

    return k


def kernel(x, indices, block_size, block_stride, block_offset):
    N, C, H, W = x.shape
    nB = indices.shape[0]
    wc = W // BW
    rows_total = N * C * H * wc

    n = indices[:, 0]
    ys = indices[:, 1] * block_stride[0] + block_offset[0]
    xs = indices[:, 2] * block_stride[1] + block_offset[1]

    h = ys[:, None] + jnp.arange(BH, dtype=jnp.int32)[None, :]
    tr = h >> 3
    hi = h & 7
    tc = (xs >> 7)[:, None]
    ck = ((xs & 127) >> 4)[:, None]
    prow16 = tr * ((W // 128) * 64) + hi * 8 + tc * 64 + ck
    plane = (n[:, None, None] * C
             + jnp.arange(C, dtype=jnp.int32)[None, None, :])
    idx_all = (plane * (H * wc)
               + prow16[:, :, None]).astype(jnp.int32)

    table = (x.reshape(N * C, H // 8, 8, W // 128, 128)
             .transpose(0, 1, 3, 2, 4)
             .reshape(rows_total, BW))
    idx_all = idx_all.reshape(nB, 2, HH * C)
    out = _make_sc_gather(nB, C, rows_total)(table, idx_all)
    return out.reshape(nB, BH, BW, C)
